# baseline (speedup 1.0000x reference)
import functools

import jax
import jax.numpy as jnp
from jax import lax
from jax.experimental import pallas as pl
from jax.experimental.pallas import tpu as pltpu
from jax.experimental.pallas import tpu_sc as plsc

N = 10000
E = 320000
NP = 10240
GPAD = NP - 1
SPAD = 10224
NW = 32
EPW = E // NW
CH = 64
NCH = 160
PH = 40
EPWP = NCH * CH
ROWS_PER_TILE = NP // 16

_MESH = plsc.VectorSubcoreMesh(core_axis_name="c", subcore_axis_name="s")


@functools.partial(
    pl.kernel,
    out_type=jax.ShapeDtypeStruct((4, 2, NP, 128), jnp.float32),
    mesh=_MESH,
    scratch_types=[
        pltpu.VMEM((NCH, CH), jnp.int32),
        pltpu.VMEM((CH, 128), jnp.float32),
        pltpu.VMEM((CH, 128), jnp.float32),
        pltpu.VMEM_SHARED((NP, 128), jnp.float32),
        pltpu.SemaphoreType.DMA,
    ],
)
def _deg_kernel(didx_hbm, out_hbm, idx_v, ones_rows, zero_rows, acc, dsem):
    c = lax.axis_index("c")
    s = lax.axis_index("s")
    w = c * 16 + s
    ones16 = jnp.full((16,), 1.0 / 128.0, dtype=jnp.float32)
    zero16 = jnp.zeros((16,), dtype=jnp.float32)

    def fill(i, _):
        for l in range(8):
            ones_rows[i, pl.ds(l * 16, 16)] = ones16
            zero_rows[i, pl.ds(l * 16, 16)] = zero16
        return 0

    lax.fori_loop(0, CH, fill, 0)
    K = 8
    for a in range(4):
        pltpu.sync_copy(didx_hbm.at[a, w], idx_v)
        for z in range(ROWS_PER_TILE // CH):
            pltpu.sync_copy(zero_rows, acc.at[pl.ds(s * ROWS_PER_TILE + z * CH, CH)])
        plsc.subcore_barrier()

        def body(gi, _):
            base = gi * K
            for b in range(K):
                pltpu.make_async_copy(
                    ones_rows, acc.at[idx_v.at[base + b]], dsem
                ).start(add=True)
            for b in range(K):
                pltpu.make_async_copy(
                    ones_rows, acc.at[idx_v.at[base + b]], dsem
                ).wait()
            return 0

        lax.fori_loop(0, NCH // K, body, 0)
        plsc.subcore_barrier()
        pltpu.sync_copy(
            acc.at[pl.ds(s * ROWS_PER_TILE, ROWS_PER_TILE)],
            out_hbm.at[a, c, pl.ds(s * ROWS_PER_TILE, ROWS_PER_TILE)],
        )
        if a + 1 < 4:
            plsc.subcore_barrier()


def _make_agg(H):
    @functools.partial(
        pl.kernel,
        out_type=jax.ShapeDtypeStruct((2, H, NP, 128), jnp.float32),
        mesh=_MESH,
        scratch_types=[
            pltpu.VMEM((PH, CH), jnp.int32),
            pltpu.VMEM((PH, CH), jnp.int32),
            pltpu.VMEM((4, CH, 128), jnp.float32),
            pltpu.VMEM_SHARED((NP, 128), jnp.float32),
            pltpu.SemaphoreType.DMA,
            pltpu.SemaphoreType.DMA,
            pltpu.SemaphoreType.DMA,
            pltpu.SemaphoreType.DMA,
            pltpu.SemaphoreType.DMA,
            pltpu.SemaphoreType.DMA,
            pltpu.SemaphoreType.DMA,
            pltpu.SemaphoreType.DMA,
        ],
    )
    def agg(src_hbm, dst_hbm, *rest):
        tables = rest[:H]
        out_hbm = rest[H]
        (src_v, dst_v, bufs, acc,
         g0, g1, g2, g3, s0, s1, s2, s3) = rest[H + 1:]
        gsems = (g0, g1, g2, g3)
        ssems = (s0, s1, s2, s3)
        c = lax.axis_index("c")
        s = lax.axis_index("s")
        w = c * 16 + s
        zero16 = jnp.zeros((16,), dtype=jnp.float32)
        for h in range(H):
            def zbody(i, _):
                for l in range(8):
                    bufs[0, i, pl.ds(l * 16, 16)] = zero16
                return 0

            lax.fori_loop(0, CH, zbody, 0)
            for z in range(ROWS_PER_TILE // CH):
                pltpu.sync_copy(bufs.at[0], acc.at[pl.ds(s * ROWS_PER_TILE + z * CH, CH)])
            plsc.subcore_barrier()

            table = tables[h]

            def g_desc(j, b):
                return pltpu.make_async_copy(table.at[src_v.at[j]], bufs.at[b], gsems[b])

            def s_desc(j, b):
                return pltpu.make_async_copy(bufs.at[b], acc.at[dst_v.at[j]], ssems[b])

            for p in range(NCH // PH):
                pltpu.sync_copy(src_hbm.at[w, pl.ds(p * PH, PH)], src_v)
                pltpu.sync_copy(dst_hbm.at[w, pl.ds(p * PH, PH)], dst_v)
                g_desc(0, 0).start()
                g_desc(1, 1).start()

                def quad(q, _):
                    for b in range(4):
                        j = 4 * q + b
                        g_desc(j, b).wait()
                        s_desc(j, b).start(add=True)

                        @pl.when(j >= 2)
                        def _():
                            s_desc(j - 2, (b + 2) % 4).wait()

                        @pl.when(j + 2 < PH)
                        def _():
                            g_desc(j + 2, (b + 2) % 4).start()

                    return 0

                lax.fori_loop(0, PH // 4, quad, 0)
                s_desc(PH - 2, (PH - 2) % 4).wait()
                s_desc(PH - 1, (PH - 1) % 4).wait()
            plsc.subcore_barrier()
            pltpu.sync_copy(
                acc.at[pl.ds(s * ROWS_PER_TILE, ROWS_PER_TILE)],
                out_hbm.at[c, h, pl.ds(s * ROWS_PER_TILE, ROWS_PER_TILE)],
            )
            if h + 1 < H:
                plsc.subcore_barrier()

    return agg


_agg1 = _make_agg(1)
_agg2 = _make_agg(2)


def _norms_tc(hists):
    RB = 1024

    def body(h_ref, o_ref):
        x = h_ref[...]
        cols = []
        for a in range(4):
            v = jnp.sum(x[a], axis=(0, 2))
            v = jnp.where(v > 0.5, lax.rsqrt(v), 0.0)
            cols.append(v[:, None])
        o_ref[...] = jnp.concatenate(cols, axis=1)

    return pl.pallas_call(
        body,
        grid=(NP // RB,),
        in_specs=[pl.BlockSpec((4, 2, RB, 128), lambda i: (0, 0, i, 0))],
        out_specs=pl.BlockSpec((RB, 4), lambda i: (i, 0)),
        out_shape=jax.ShapeDtypeStruct((NP, 4), jnp.float32),
    )(hists)


def _scale_tc(x, ns):
    RB = 1024

    def body(x_ref, n_ref, o_ref):
        o_ref[...] = x_ref[...] * n_ref[...]

    return pl.pallas_call(
        body,
        grid=(NP // RB,),
        in_specs=[
            pl.BlockSpec((RB, 128), lambda i: (i, 0)),
            pl.BlockSpec((RB, 1), lambda i: (i, 0)),
        ],
        out_specs=pl.BlockSpec((RB, 128), lambda i: (i, 0)),
        out_shape=jax.ShapeDtypeStruct((NP, 128), jnp.float32),
    )(x, ns)


def _layer_tc(P, nd, ns, W):
    H = P.shape[1]
    RB = 1024

    def body(p_ref, nd_ref, ns_ref, w_ref, o0_ref, o1_ref):
        parts = [p_ref[0, h] + p_ref[1, h] for h in range(H)]
        X = parts[0] if H == 1 else jnp.concatenate(parts, axis=1)
        X = X * nd_ref[...]
        Y = jnp.dot(X, w_ref[...], preferred_element_type=jnp.float32)
        Y = jnp.maximum(Y, 0.0) * ns_ref[...]
        o0_ref[...] = Y[:, :128]
        o1_ref[...] = Y[:, 128:]

    return pl.pallas_call(
        body,
        grid=(NP // RB,),
        in_specs=[
            pl.BlockSpec((2, H, RB, 128), lambda i: (0, 0, i, 0)),
            pl.BlockSpec((RB, 1), lambda i: (i, 0)),
            pl.BlockSpec((RB, 1), lambda i: (i, 0)),
            pl.BlockSpec((128 * H, 256), lambda i: (0, 0)),
        ],
        out_specs=[
            pl.BlockSpec((RB, 128), lambda i: (i, 0)),
            pl.BlockSpec((RB, 128), lambda i: (i, 0)),
        ],
        out_shape=[
            jax.ShapeDtypeStruct((NP, 128), jnp.float32),
            jax.ShapeDtypeStruct((NP, 128), jnp.float32),
        ],
    )(P, nd, ns, W)


def _head_tc(P, nd, Wm, Ws):
    RB = 1024

    def body(p_ref, nd_ref, wm_ref, ws_ref, om_ref, os_ref):
        X = jnp.concatenate([p_ref[0, 0] + p_ref[1, 0], p_ref[0, 1] + p_ref[1, 1]], axis=1)
        X = X * nd_ref[...]
        om_ref[...] = jnp.dot(X, wm_ref[...], preferred_element_type=jnp.float32)
        os_ref[...] = jnp.dot(X, ws_ref[...], preferred_element_type=jnp.float32)

    return pl.pallas_call(
        body,
        grid=(NP // RB,),
        in_specs=[
            pl.BlockSpec((2, 2, RB, 128), lambda i: (0, 0, i, 0)),
            pl.BlockSpec((RB, 1), lambda i: (i, 0)),
            pl.BlockSpec((256, 64), lambda i: (0, 0)),
            pl.BlockSpec((256, 64), lambda i: (0, 0)),
        ],
        out_specs=[
            pl.BlockSpec((RB, 64), lambda i: (i, 0)),
            pl.BlockSpec((RB, 64), lambda i: (i, 0)),
        ],
        out_shape=[
            jax.ShapeDtypeStruct((NP, 64), jnp.float32),
            jax.ShapeDtypeStruct((NP, 64), jnp.float32),
        ],
    )(P, nd, Wm, Ws)


def _final_tc(A_mean, A_std, noise_A, S_mean, S_std, noise_S, Wza, Wzb):
    def body(am, ast, na, sm, sst, nsn, wa, wb, z1_ref, z2_ref):
        def one(mean_ref, std_ref, noise_ref, w_ref, out_ref):
            z = mean_ref[...] + noise_ref[...] * jnp.exp(std_ref[...])
            m = jnp.mean(z, axis=0, keepdims=True)
            d = z - m
            var = jnp.sum(d * d, axis=0, keepdims=True) / (N - 1)
            zn = d * lax.rsqrt(var)
            out_ref[...] = lax.dot_general(
                zn, w_ref[...], (((1,), (1,)), ((), ())),
                preferred_element_type=jnp.float32,
            )

        one(am, ast, na, wa, z1_ref)
        one(sm, sst, nsn, wb, z2_ref)

    return pl.pallas_call(
        body,
        out_shape=[
            jax.ShapeDtypeStruct((N, 64), jnp.float32),
            jax.ShapeDtypeStruct((N, 64), jnp.float32),
        ],
    )(A_mean, A_std, noise_A, S_mean, S_std, noise_S, Wza, Wzb)


def _prep_edges(ei):
    src = ei[0].astype(jnp.int32).reshape(NW, EPW)
    dst = ei[1].astype(jnp.int32).reshape(NW, EPW)
    pad = EPWP - EPW
    src = jnp.pad(src, ((0, 0), (0, pad)), constant_values=GPAD).reshape(NW, NCH, CH)
    dst = jnp.pad(dst, ((0, 0), (0, pad)), constant_values=SPAD).reshape(NW, NCH, CH)
    return src, dst


def kernel(g, adj, features, add_features, Wk0, Wk1, Wk2, Wh0, Wh1, Wh2, Wstd, Wza, Wzb):
    asrc, adst = _prep_edges(adj)
    gsrc, gdst = _prep_edges(g)

    pad = EPWP - EPW

    def dpad(v):
        return jnp.pad(
            v.astype(jnp.int32).reshape(NW, EPW),
            ((0, 0), (0, pad)),
            constant_values=SPAD,
        ).reshape(NW, NCH, CH)

    didx = jnp.stack([dpad(adj[0]), dpad(adj[1]), dpad(g[0]), dpad(g[1])])
    hists = _deg_kernel(didx)
    norms_t = _norms_tc(hists)
    a_ns, a_nd = norms_t[:, 0:1], norms_t[:, 1:2]
    g_ns, g_nd = norms_t[:, 2:3], norms_t[:, 3:4]

    featp = jnp.pad(features, ((0, NP - N), (0, 0)))
    addfp = jnp.pad(add_features, ((0, NP - N), (0, 0)))

    def run_stack(src, dst, ns, nd, x0, W0, W1, W2, Wstd_):
        x0t = _scale_tc(x0, ns)
        P0 = _agg1(src, dst, x0t)
        h1a, h1b = _layer_tc(P0, nd, ns, W0)
        P1 = _agg2(src, dst, h1a, h1b)
        h2a, h2b = _layer_tc(P1, nd, ns, W1)
        P2 = _agg2(src, dst, h2a, h2b)
        mean, std = _head_tc(P2, nd, W2, Wstd_)
        return mean[:N], std[:N]

    A_mean, A_std = run_stack(asrc, adst, a_ns, a_nd, featp, Wk0, Wk1, Wk2, Wstd)
    S_mean, S_std = run_stack(gsrc, gdst, g_ns, g_nd, addfp, Wh0, Wh1, Wh2, Wstd)

    nk = jax.random.key(42)
    noise_A = jax.random.normal(jax.random.fold_in(nk, 0), (N, 64), dtype=jnp.float32)
    noise_S = jax.random.normal(jax.random.fold_in(nk, 1), (N, 64), dtype=jnp.float32)

    z1, z2 = _final_tc(A_mean, A_std, noise_A, S_mean, S_std, noise_S, Wza, Wzb)
    return (z1, z2, A_mean, S_mean, A_std, S_std)


# speedup vs baseline: 1.0005x; 1.0005x over previous
import functools

import jax
import jax.numpy as jnp
from jax import lax
from jax.experimental import pallas as pl
from jax.experimental.pallas import tpu as pltpu
from jax.experimental.pallas import tpu_sc as plsc

N = 10000
E = 320000
NP = 10240
GPAD = NP - 1
SPAD = 10224
NW = 32
EPW = E // NW
CH = 128
NCH = 80
PH = 40
EPWP = NCH * CH
ROWS_PER_TILE = NP // 16

_MESH = plsc.VectorSubcoreMesh(core_axis_name="c", subcore_axis_name="s")


@functools.partial(
    pl.kernel,
    out_type=jax.ShapeDtypeStruct((4, 2, NP, 128), jnp.float32),
    mesh=_MESH,
    scratch_types=[
        pltpu.VMEM((NCH, CH), jnp.int32),
        pltpu.VMEM((CH, 128), jnp.float32),
        pltpu.VMEM((CH, 128), jnp.float32),
        pltpu.VMEM_SHARED((NP, 128), jnp.float32),
        pltpu.SemaphoreType.DMA,
    ],
)
def _deg_kernel(didx_hbm, out_hbm, idx_v, ones_rows, zero_rows, acc, dsem):
    c = lax.axis_index("c")
    s = lax.axis_index("s")
    w = c * 16 + s
    ones16 = jnp.full((16,), 1.0 / 128.0, dtype=jnp.float32)
    zero16 = jnp.zeros((16,), dtype=jnp.float32)

    def fill(i, _):
        for l in range(8):
            ones_rows[i, pl.ds(l * 16, 16)] = ones16
            zero_rows[i, pl.ds(l * 16, 16)] = zero16
        return 0

    lax.fori_loop(0, CH, fill, 0)
    K = 8
    for a in range(4):
        pltpu.sync_copy(didx_hbm.at[a, w], idx_v)
        for z in range(ROWS_PER_TILE // CH):
            pltpu.sync_copy(zero_rows, acc.at[pl.ds(s * ROWS_PER_TILE + z * CH, CH)])
        plsc.subcore_barrier()

        def body(gi, _):
            base = gi * K
            for b in range(K):
                pltpu.make_async_copy(
                    ones_rows, acc.at[idx_v.at[base + b]], dsem
                ).start(add=True)
            for b in range(K):
                pltpu.make_async_copy(
                    ones_rows, acc.at[idx_v.at[base + b]], dsem
                ).wait()
            return 0

        lax.fori_loop(0, NCH // K, body, 0)
        plsc.subcore_barrier()
        pltpu.sync_copy(
            acc.at[pl.ds(s * ROWS_PER_TILE, ROWS_PER_TILE)],
            out_hbm.at[a, c, pl.ds(s * ROWS_PER_TILE, ROWS_PER_TILE)],
        )
        if a + 1 < 4:
            plsc.subcore_barrier()


def _make_agg(H):
    @functools.partial(
        pl.kernel,
        out_type=jax.ShapeDtypeStruct((2, H, NP, 128), jnp.float32),
        mesh=_MESH,
        scratch_types=[
            pltpu.VMEM((PH, CH), jnp.int32),
            pltpu.VMEM((PH, CH), jnp.int32),
            pltpu.VMEM((2, CH, 128), jnp.float32),
            pltpu.VMEM_SHARED((NP, 128), jnp.float32),
            pltpu.SemaphoreType.DMA,
            pltpu.SemaphoreType.DMA,
            pltpu.SemaphoreType.DMA,
            pltpu.SemaphoreType.DMA,
        ],
    )
    def agg(src_hbm, dst_hbm, *rest):
        tables = rest[:H]
        out_hbm = rest[H]
        (src_v, dst_v, bufs, acc, g0, g1, s0, s1) = rest[H + 1:]
        gsems = (g0, g1)
        ssems = (s0, s1)
        c = lax.axis_index("c")
        s = lax.axis_index("s")
        w = c * 16 + s
        zero16 = jnp.zeros((16,), dtype=jnp.float32)
        for h in range(H):
            def zbody(i, _):
                for l in range(8):
                    bufs[0, i, pl.ds(l * 16, 16)] = zero16
                return 0

            lax.fori_loop(0, CH, zbody, 0)
            for z in range(ROWS_PER_TILE // CH):
                pltpu.sync_copy(bufs.at[0], acc.at[pl.ds(s * ROWS_PER_TILE + z * CH, CH)])
            plsc.subcore_barrier()

            table = tables[h]

            def g_desc(j, b):
                return pltpu.make_async_copy(table.at[src_v.at[j]], bufs.at[b], gsems[b])

            def s_desc(j, b):
                return pltpu.make_async_copy(bufs.at[b], acc.at[dst_v.at[j]], ssems[b])

            for p in range(NCH // PH):
                pltpu.sync_copy(src_hbm.at[w, pl.ds(p * PH, PH)], src_v)
                pltpu.sync_copy(dst_hbm.at[w, pl.ds(p * PH, PH)], dst_v)
                g_desc(0, 0).start()

                def pair(q, _):
                    for b in range(2):
                        j = 2 * q + b
                        g_desc(j, b).wait()
                        s_desc(j, b).start(add=True)

                        @pl.when(j >= 1)
                        def _():
                            s_desc(j - 1, 1 - b).wait()

                        @pl.when(j + 1 < PH)
                        def _():
                            g_desc(j + 1, 1 - b).start()

                    return 0

                lax.fori_loop(0, PH // 2, pair, 0)
                s_desc(PH - 1, (PH - 1) % 2).wait()
            plsc.subcore_barrier()
            pltpu.sync_copy(
                acc.at[pl.ds(s * ROWS_PER_TILE, ROWS_PER_TILE)],
                out_hbm.at[c, h, pl.ds(s * ROWS_PER_TILE, ROWS_PER_TILE)],
            )
            if h + 1 < H:
                plsc.subcore_barrier()

    return agg


_agg1 = _make_agg(1)
_agg2 = _make_agg(2)


def _norms_tc(hists):
    RB = 1024

    def body(h_ref, o_ref):
        x = h_ref[...]
        cols = []
        for a in range(4):
            v = jnp.sum(x[a], axis=(0, 2))
            v = jnp.where(v > 0.5, lax.rsqrt(v), 0.0)
            cols.append(v[:, None])
        o_ref[...] = jnp.concatenate(cols, axis=1)

    return pl.pallas_call(
        body,
        grid=(NP // RB,),
        in_specs=[pl.BlockSpec((4, 2, RB, 128), lambda i: (0, 0, i, 0))],
        out_specs=pl.BlockSpec((RB, 4), lambda i: (i, 0)),
        out_shape=jax.ShapeDtypeStruct((NP, 4), jnp.float32),
    )(hists)


def _scale_tc(x, ns):
    RB = 1024

    def body(x_ref, n_ref, o_ref):
        o_ref[...] = x_ref[...] * n_ref[...]

    return pl.pallas_call(
        body,
        grid=(NP // RB,),
        in_specs=[
            pl.BlockSpec((RB, 128), lambda i: (i, 0)),
            pl.BlockSpec((RB, 1), lambda i: (i, 0)),
        ],
        out_specs=pl.BlockSpec((RB, 128), lambda i: (i, 0)),
        out_shape=jax.ShapeDtypeStruct((NP, 128), jnp.float32),
    )(x, ns)


def _layer_tc(P, nd, ns, W):
    H = P.shape[1]
    RB = 1024

    def body(p_ref, nd_ref, ns_ref, w_ref, o0_ref, o1_ref):
        parts = [p_ref[0, h] + p_ref[1, h] for h in range(H)]
        X = parts[0] if H == 1 else jnp.concatenate(parts, axis=1)
        X = X * nd_ref[...]
        Y = jnp.dot(X, w_ref[...], preferred_element_type=jnp.float32)
        Y = jnp.maximum(Y, 0.0) * ns_ref[...]
        o0_ref[...] = Y[:, :128]
        o1_ref[...] = Y[:, 128:]

    return pl.pallas_call(
        body,
        grid=(NP // RB,),
        in_specs=[
            pl.BlockSpec((2, H, RB, 128), lambda i: (0, 0, i, 0)),
            pl.BlockSpec((RB, 1), lambda i: (i, 0)),
            pl.BlockSpec((RB, 1), lambda i: (i, 0)),
            pl.BlockSpec((128 * H, 256), lambda i: (0, 0)),
        ],
        out_specs=[
            pl.BlockSpec((RB, 128), lambda i: (i, 0)),
            pl.BlockSpec((RB, 128), lambda i: (i, 0)),
        ],
        out_shape=[
            jax.ShapeDtypeStruct((NP, 128), jnp.float32),
            jax.ShapeDtypeStruct((NP, 128), jnp.float32),
        ],
    )(P, nd, ns, W)


def _head_tc(P, nd, Wm, Ws):
    RB = 1024

    def body(p_ref, nd_ref, wm_ref, ws_ref, om_ref, os_ref):
        X = jnp.concatenate([p_ref[0, 0] + p_ref[1, 0], p_ref[0, 1] + p_ref[1, 1]], axis=1)
        X = X * nd_ref[...]
        om_ref[...] = jnp.dot(X, wm_ref[...], preferred_element_type=jnp.float32)
        os_ref[...] = jnp.dot(X, ws_ref[...], preferred_element_type=jnp.float32)

    return pl.pallas_call(
        body,
        grid=(NP // RB,),
        in_specs=[
            pl.BlockSpec((2, 2, RB, 128), lambda i: (0, 0, i, 0)),
            pl.BlockSpec((RB, 1), lambda i: (i, 0)),
            pl.BlockSpec((256, 64), lambda i: (0, 0)),
            pl.BlockSpec((256, 64), lambda i: (0, 0)),
        ],
        out_specs=[
            pl.BlockSpec((RB, 64), lambda i: (i, 0)),
            pl.BlockSpec((RB, 64), lambda i: (i, 0)),
        ],
        out_shape=[
            jax.ShapeDtypeStruct((NP, 64), jnp.float32),
            jax.ShapeDtypeStruct((NP, 64), jnp.float32),
        ],
    )(P, nd, Wm, Ws)


def _final_tc(A_mean, A_std, noise_A, S_mean, S_std, noise_S, Wza, Wzb):
    def body(am, ast, na, sm, sst, nsn, wa, wb, z1_ref, z2_ref):
        def one(mean_ref, std_ref, noise_ref, w_ref, out_ref):
            z = mean_ref[...] + noise_ref[...] * jnp.exp(std_ref[...])
            m = jnp.mean(z, axis=0, keepdims=True)
            d = z - m
            var = jnp.sum(d * d, axis=0, keepdims=True) / (N - 1)
            zn = d * lax.rsqrt(var)
            out_ref[...] = lax.dot_general(
                zn, w_ref[...], (((1,), (1,)), ((), ())),
                preferred_element_type=jnp.float32,
            )

        one(am, ast, na, wa, z1_ref)
        one(sm, sst, nsn, wb, z2_ref)

    return pl.pallas_call(
        body,
        out_shape=[
            jax.ShapeDtypeStruct((N, 64), jnp.float32),
            jax.ShapeDtypeStruct((N, 64), jnp.float32),
        ],
    )(A_mean, A_std, noise_A, S_mean, S_std, noise_S, Wza, Wzb)


def _prep_edges(ei):
    src = ei[0].astype(jnp.int32).reshape(NW, EPW)
    dst = ei[1].astype(jnp.int32).reshape(NW, EPW)
    pad = EPWP - EPW
    src = jnp.pad(src, ((0, 0), (0, pad)), constant_values=GPAD).reshape(NW, NCH, CH)
    dst = jnp.pad(dst, ((0, 0), (0, pad)), constant_values=SPAD).reshape(NW, NCH, CH)
    return src, dst


def kernel(g, adj, features, add_features, Wk0, Wk1, Wk2, Wh0, Wh1, Wh2, Wstd, Wza, Wzb):
    asrc, adst = _prep_edges(adj)
    gsrc, gdst = _prep_edges(g)

    pad = EPWP - EPW

    def dpad(v):
        return jnp.pad(
            v.astype(jnp.int32).reshape(NW, EPW),
            ((0, 0), (0, pad)),
            constant_values=SPAD,
        ).reshape(NW, NCH, CH)

    didx = jnp.stack([dpad(adj[0]), dpad(adj[1]), dpad(g[0]), dpad(g[1])])
    hists = _deg_kernel(didx)
    norms_t = _norms_tc(hists)
    a_ns, a_nd = norms_t[:, 0:1], norms_t[:, 1:2]
    g_ns, g_nd = norms_t[:, 2:3], norms_t[:, 3:4]

    featp = jnp.pad(features, ((0, NP - N), (0, 0)))
    addfp = jnp.pad(add_features, ((0, NP - N), (0, 0)))

    def run_stack(src, dst, ns, nd, x0, W0, W1, W2, Wstd_):
        x0t = _scale_tc(x0, ns)
        P0 = _agg1(src, dst, x0t)
        h1a, h1b = _layer_tc(P0, nd, ns, W0)
        P1 = _agg2(src, dst, h1a, h1b)
        h2a, h2b = _layer_tc(P1, nd, ns, W1)
        P2 = _agg2(src, dst, h2a, h2b)
        mean, std = _head_tc(P2, nd, W2, Wstd_)
        return mean[:N], std[:N]

    A_mean, A_std = run_stack(asrc, adst, a_ns, a_nd, featp, Wk0, Wk1, Wk2, Wstd)
    S_mean, S_std = run_stack(gsrc, gdst, g_ns, g_nd, addfp, Wh0, Wh1, Wh2, Wstd)

    nk = jax.random.key(42)
    noise_A = jax.random.normal(jax.random.fold_in(nk, 0), (N, 64), dtype=jnp.float32)
    noise_S = jax.random.normal(jax.random.fold_in(nk, 1), (N, 64), dtype=jnp.float32)

    z1, z2 = _final_tc(A_mean, A_std, noise_A, S_mean, S_std, noise_S, Wza, Wzb)
    return (z1, z2, A_mean, S_mean, A_std, S_std)


# speedup vs baseline: 1.1186x; 1.1180x over previous
import functools

import jax
import jax.numpy as jnp
from jax import lax
from jax.experimental import pallas as pl
from jax.experimental.pallas import tpu as pltpu
from jax.experimental.pallas import tpu_sc as plsc

N = 10000
E = 320000
NP = 10240
GPAD = NP - 1
SPAD = 10224
NW = 32
EPW = E // NW
CH = 128
NCH = 80
PH = 40
EPWP = NCH * CH
ROWS_PER_TILE = NP // 16

_MESH = plsc.VectorSubcoreMesh(core_axis_name="c", subcore_axis_name="s")


@functools.partial(
    pl.kernel,
    out_type=jax.ShapeDtypeStruct((4, 2, NP, 128), jnp.float32),
    mesh=_MESH,
    scratch_types=[
        pltpu.VMEM((NCH, CH), jnp.int32),
        pltpu.VMEM((CH, 128), jnp.float32),
        pltpu.VMEM((CH, 128), jnp.float32),
        pltpu.VMEM_SHARED((NP, 128), jnp.float32),
        pltpu.SemaphoreType.DMA,
    ],
)
def _deg_kernel(didx_hbm, out_hbm, idx_v, ones_rows, zero_rows, acc, dsem):
    c = lax.axis_index("c")
    s = lax.axis_index("s")
    w = c * 16 + s
    ones16 = jnp.full((16,), 1.0 / 128.0, dtype=jnp.float32)
    zero16 = jnp.zeros((16,), dtype=jnp.float32)

    def fill(i, _):
        for l in range(8):
            ones_rows[i, pl.ds(l * 16, 16)] = ones16
            zero_rows[i, pl.ds(l * 16, 16)] = zero16
        return 0

    lax.fori_loop(0, CH, fill, 0)
    K = 8
    for a in range(4):
        pltpu.sync_copy(didx_hbm.at[a, w], idx_v)
        for z in range(ROWS_PER_TILE // CH):
            pltpu.sync_copy(zero_rows, acc.at[pl.ds(s * ROWS_PER_TILE + z * CH, CH)])
        plsc.subcore_barrier()

        def body(gi, _):
            base = gi * K
            for b in range(K):
                pltpu.make_async_copy(
                    ones_rows, acc.at[idx_v.at[base + b]], dsem
                ).start(add=True)
            for b in range(K):
                pltpu.make_async_copy(
                    ones_rows, acc.at[idx_v.at[base + b]], dsem
                ).wait()
            return 0

        lax.fori_loop(0, NCH // K, body, 0)
        plsc.subcore_barrier()
        pltpu.sync_copy(
            acc.at[pl.ds(s * ROWS_PER_TILE, ROWS_PER_TILE)],
            out_hbm.at[a, c, pl.ds(s * ROWS_PER_TILE, ROWS_PER_TILE)],
        )
        if a + 1 < 4:
            plsc.subcore_barrier()


def _make_agg(H):
    @functools.partial(
        pl.kernel,
        out_type=jax.ShapeDtypeStruct((2, H, NP, 128), jnp.float32),
        mesh=_MESH,
        scratch_types=[
            pltpu.VMEM((PH, CH), jnp.int32),
            pltpu.VMEM((PH, CH), jnp.int32),
            pltpu.VMEM((2, CH, 128), jnp.float32),
            pltpu.VMEM_SHARED((NP, 128), jnp.float32),
            pltpu.SemaphoreType.DMA,
            pltpu.SemaphoreType.DMA,
            pltpu.SemaphoreType.DMA,
            pltpu.SemaphoreType.DMA,
        ],
    )
    def agg(src_hbm, dst_hbm, *rest):
        tables = rest[:H]
        out_hbm = rest[H]
        (src_v, dst_v, bufs, acc, g0, g1, s0, s1) = rest[H + 1:]
        gsems = (g0, g1)
        ssems = (s0, s1)
        c = lax.axis_index("c")
        s = lax.axis_index("s")
        w = c * 16 + s
        zero16 = jnp.zeros((16,), dtype=jnp.float32)
        for h in range(H):
            def zbody(i, _):
                for l in range(8):
                    bufs[0, i, pl.ds(l * 16, 16)] = zero16
                return 0

            lax.fori_loop(0, CH, zbody, 0)
            for z in range(ROWS_PER_TILE // CH):
                pltpu.sync_copy(bufs.at[0], acc.at[pl.ds(s * ROWS_PER_TILE + z * CH, CH)])
            plsc.subcore_barrier()

            table = tables[h]

            def g_desc(j, b):
                return pltpu.make_async_copy(table.at[src_v.at[j]], bufs.at[b], gsems[b])

            def s_desc(j, b):
                return pltpu.make_async_copy(bufs.at[b], acc.at[dst_v.at[j]], ssems[b])

            for p in range(NCH // PH):
                pltpu.sync_copy(src_hbm.at[w, pl.ds(p * PH, PH)], src_v)
                pltpu.sync_copy(dst_hbm.at[w, pl.ds(p * PH, PH)], dst_v)

                def body(j, _):
                    g_desc(j, 0).start()
                    g_desc(j, 0).wait()
                    s_desc(j, 0).start(add=True)
                    s_desc(j, 0).wait()
                    return 0

                lax.fori_loop(0, PH, body, 0)
            plsc.subcore_barrier()
            pltpu.sync_copy(
                acc.at[pl.ds(s * ROWS_PER_TILE, ROWS_PER_TILE)],
                out_hbm.at[c, h, pl.ds(s * ROWS_PER_TILE, ROWS_PER_TILE)],
            )
            if h + 1 < H:
                plsc.subcore_barrier()

    return agg


_agg1 = _make_agg(1)
_agg2 = _make_agg(2)


def _norms_tc(hists):
    RB = 1024

    def body(h_ref, o_ref):
        x = h_ref[...]
        cols = []
        for a in range(4):
            v = jnp.sum(x[a], axis=(0, 2))
            v = jnp.where(v > 0.5, lax.rsqrt(v), 0.0)
            cols.append(v[:, None])
        o_ref[...] = jnp.concatenate(cols, axis=1)

    return pl.pallas_call(
        body,
        grid=(NP // RB,),
        in_specs=[pl.BlockSpec((4, 2, RB, 128), lambda i: (0, 0, i, 0))],
        out_specs=pl.BlockSpec((RB, 4), lambda i: (i, 0)),
        out_shape=jax.ShapeDtypeStruct((NP, 4), jnp.float32),
    )(hists)


def _scale_tc(x, ns):
    RB = 1024

    def body(x_ref, n_ref, o_ref):
        o_ref[...] = x_ref[...] * n_ref[...]

    return pl.pallas_call(
        body,
        grid=(NP // RB,),
        in_specs=[
            pl.BlockSpec((RB, 128), lambda i: (i, 0)),
            pl.BlockSpec((RB, 1), lambda i: (i, 0)),
        ],
        out_specs=pl.BlockSpec((RB, 128), lambda i: (i, 0)),
        out_shape=jax.ShapeDtypeStruct((NP, 128), jnp.float32),
    )(x, ns)


def _layer_tc(P, nd, ns, W):
    H = P.shape[1]
    RB = 1024

    def body(p_ref, nd_ref, ns_ref, w_ref, o0_ref, o1_ref):
        parts = [p_ref[0, h] + p_ref[1, h] for h in range(H)]
        X = parts[0] if H == 1 else jnp.concatenate(parts, axis=1)
        X = X * nd_ref[...]
        Y = jnp.dot(X, w_ref[...], preferred_element_type=jnp.float32)
        Y = jnp.maximum(Y, 0.0) * ns_ref[...]
        o0_ref[...] = Y[:, :128]
        o1_ref[...] = Y[:, 128:]

    return pl.pallas_call(
        body,
        grid=(NP // RB,),
        in_specs=[
            pl.BlockSpec((2, H, RB, 128), lambda i: (0, 0, i, 0)),
            pl.BlockSpec((RB, 1), lambda i: (i, 0)),
            pl.BlockSpec((RB, 1), lambda i: (i, 0)),
            pl.BlockSpec((128 * H, 256), lambda i: (0, 0)),
        ],
        out_specs=[
            pl.BlockSpec((RB, 128), lambda i: (i, 0)),
            pl.BlockSpec((RB, 128), lambda i: (i, 0)),
        ],
        out_shape=[
            jax.ShapeDtypeStruct((NP, 128), jnp.float32),
            jax.ShapeDtypeStruct((NP, 128), jnp.float32),
        ],
    )(P, nd, ns, W)


def _layer2_head_tc(P, nd, ns, W1, Wm, Ws):
    RB = 1024

    def body(p_ref, nd_ref, ns_ref, w1_ref, wm_ref, ws_ref, o_ref):
        X = jnp.concatenate([p_ref[0, 0] + p_ref[1, 0], p_ref[0, 1] + p_ref[1, 1]], axis=1)
        X = X * nd_ref[...]
        h2 = jnp.dot(X, w1_ref[...], preferred_element_type=jnp.float32)
        h2 = jnp.maximum(h2, 0.0) * ns_ref[...]
        tm = jnp.dot(h2, wm_ref[...], preferred_element_type=jnp.float32)
        ts = jnp.dot(h2, ws_ref[...], preferred_element_type=jnp.float32)
        o_ref[...] = jnp.concatenate([tm, ts], axis=1)

    return pl.pallas_call(
        body,
        grid=(NP // RB,),
        in_specs=[
            pl.BlockSpec((2, 2, RB, 128), lambda i: (0, 0, i, 0)),
            pl.BlockSpec((RB, 1), lambda i: (i, 0)),
            pl.BlockSpec((RB, 1), lambda i: (i, 0)),
            pl.BlockSpec((256, 256), lambda i: (0, 0)),
            pl.BlockSpec((256, 64), lambda i: (0, 0)),
            pl.BlockSpec((256, 64), lambda i: (0, 0)),
        ],
        out_specs=pl.BlockSpec((RB, 128), lambda i: (i, 0)),
        out_shape=jax.ShapeDtypeStruct((NP, 128), jnp.float32),
    )(P, nd, ns, W1, Wm, Ws)


def _head_fin_tc(P, nd):
    RB = 1024

    def body(p_ref, nd_ref, om_ref, os_ref):
        X = (p_ref[0, 0] + p_ref[1, 0]) * nd_ref[...]
        om_ref[...] = X[:, :64]
        os_ref[...] = X[:, 64:]

    return pl.pallas_call(
        body,
        grid=(NP // RB,),
        in_specs=[
            pl.BlockSpec((2, 1, RB, 128), lambda i: (0, 0, i, 0)),
            pl.BlockSpec((RB, 1), lambda i: (i, 0)),
        ],
        out_specs=[
            pl.BlockSpec((RB, 64), lambda i: (i, 0)),
            pl.BlockSpec((RB, 64), lambda i: (i, 0)),
        ],
        out_shape=[
            jax.ShapeDtypeStruct((NP, 64), jnp.float32),
            jax.ShapeDtypeStruct((NP, 64), jnp.float32),
        ],
    )(P, nd)


def _final_tc(A_mean, A_std, noise_A, S_mean, S_std, noise_S, Wza, Wzb):
    def body(am, ast, na, sm, sst, nsn, wa, wb, z1_ref, z2_ref):
        def one(mean_ref, std_ref, noise_ref, w_ref, out_ref):
            z = mean_ref[...] + noise_ref[...] * jnp.exp(std_ref[...])
            m = jnp.mean(z, axis=0, keepdims=True)
            d = z - m
            var = jnp.sum(d * d, axis=0, keepdims=True) / (N - 1)
            zn = d * lax.rsqrt(var)
            out_ref[...] = lax.dot_general(
                zn, w_ref[...], (((1,), (1,)), ((), ())),
                preferred_element_type=jnp.float32,
            )

        one(am, ast, na, wa, z1_ref)
        one(sm, sst, nsn, wb, z2_ref)

    return pl.pallas_call(
        body,
        out_shape=[
            jax.ShapeDtypeStruct((N, 64), jnp.float32),
            jax.ShapeDtypeStruct((N, 64), jnp.float32),
        ],
    )(A_mean, A_std, noise_A, S_mean, S_std, noise_S, Wza, Wzb)


def _prep_edges(ei):
    src = ei[0].astype(jnp.int32).reshape(NW, EPW)
    dst = ei[1].astype(jnp.int32).reshape(NW, EPW)
    pad = EPWP - EPW
    src = jnp.pad(src, ((0, 0), (0, pad)), constant_values=GPAD).reshape(NW, NCH, CH)
    dst = jnp.pad(dst, ((0, 0), (0, pad)), constant_values=SPAD).reshape(NW, NCH, CH)
    return src, dst


def kernel(g, adj, features, add_features, Wk0, Wk1, Wk2, Wh0, Wh1, Wh2, Wstd, Wza, Wzb):
    asrc, adst = _prep_edges(adj)
    gsrc, gdst = _prep_edges(g)

    pad = EPWP - EPW

    def dpad(v):
        return jnp.pad(
            v.astype(jnp.int32).reshape(NW, EPW),
            ((0, 0), (0, pad)),
            constant_values=SPAD,
        ).reshape(NW, NCH, CH)

    didx = jnp.stack([dpad(adj[0]), dpad(adj[1]), dpad(g[0]), dpad(g[1])])
    hists = _deg_kernel(didx)
    norms_t = _norms_tc(hists)
    a_ns, a_nd = norms_t[:, 0:1], norms_t[:, 1:2]
    g_ns, g_nd = norms_t[:, 2:3], norms_t[:, 3:4]

    featp = jnp.pad(features, ((0, NP - N), (0, 0)))
    addfp = jnp.pad(add_features, ((0, NP - N), (0, 0)))

    def run_stack(src, dst, ns, nd, x0, W0, W1, W2, Wstd_):
        x0t = _scale_tc(x0, ns)
        P0 = _agg1(src, dst, x0t)
        h1a, h1b = _layer_tc(P0, nd, ns, W0)
        P1 = _agg2(src, dst, h1a, h1b)
        T = _layer2_head_tc(P1, nd, ns, W1, W2, Wstd_)
        P2 = _agg1(src, dst, T)
        mean, std = _head_fin_tc(P2, nd)
        return mean[:N], std[:N]

    A_mean, A_std = run_stack(asrc, adst, a_ns, a_nd, featp, Wk0, Wk1, Wk2, Wstd)
    S_mean, S_std = run_stack(gsrc, gdst, g_ns, g_nd, addfp, Wh0, Wh1, Wh2, Wstd)

    nk = jax.random.key(42)
    noise_A = jax.random.normal(jax.random.fold_in(nk, 0), (N, 64), dtype=jnp.float32)
    noise_S = jax.random.normal(jax.random.fold_in(nk, 1), (N, 64), dtype=jnp.float32)

    z1, z2 = _final_tc(A_mean, A_std, noise_A, S_mean, S_std, noise_S, Wza, Wzb)
    return (z1, z2, A_mean, S_mean, A_std, S_std)


# speedup vs baseline: 1.1224x; 1.0034x over previous
import functools

import jax
import jax.numpy as jnp
from jax import lax
from jax.experimental import pallas as pl
from jax.experimental.pallas import tpu as pltpu
from jax.experimental.pallas import tpu_sc as plsc

N = 10000
E = 320000
NP = 10240
GPAD = NP - 1
SPAD = 10224
NW = 32
EPW = E // NW
CH = 128
NCH = 80
PH = 40
EPWP = NCH * CH
ROWS_PER_TILE = NP // 16

_MESH = plsc.VectorSubcoreMesh(core_axis_name="c", subcore_axis_name="s")


@functools.partial(
    pl.kernel,
    out_type=jax.ShapeDtypeStruct((4, 2, NP, 128), jnp.float32),
    mesh=_MESH,
    scratch_types=[
        pltpu.VMEM((NCH, CH), jnp.int32),
        pltpu.VMEM((CH, 128), jnp.float32),
        pltpu.VMEM((CH, 128), jnp.float32),
        pltpu.VMEM_SHARED((NP, 128), jnp.float32),
        pltpu.SemaphoreType.DMA,
    ],
)
def _deg_kernel(didx_hbm, out_hbm, idx_v, ones_rows, zero_rows, acc, dsem):
    c = lax.axis_index("c")
    s = lax.axis_index("s")
    w = c * 16 + s
    ones16 = jnp.full((16,), 1.0 / 128.0, dtype=jnp.float32)
    zero16 = jnp.zeros((16,), dtype=jnp.float32)

    def fill(i, _):
        for l in range(8):
            ones_rows[i, pl.ds(l * 16, 16)] = ones16
            zero_rows[i, pl.ds(l * 16, 16)] = zero16
        return 0

    lax.fori_loop(0, CH, fill, 0)
    K = 8
    for a in range(4):
        pltpu.sync_copy(didx_hbm.at[a, w], idx_v)
        for z in range(ROWS_PER_TILE // CH):
            pltpu.sync_copy(zero_rows, acc.at[pl.ds(s * ROWS_PER_TILE + z * CH, CH)])
        plsc.subcore_barrier()

        def body(gi, _):
            base = gi * K
            for b in range(K):
                pltpu.make_async_copy(
                    ones_rows, acc.at[idx_v.at[base + b]], dsem
                ).start(add=True)
            for b in range(K):
                pltpu.make_async_copy(
                    ones_rows, acc.at[idx_v.at[base + b]], dsem
                ).wait()
            return 0

        lax.fori_loop(0, NCH // K, body, 0)
        plsc.subcore_barrier()
        pltpu.sync_copy(
            acc.at[pl.ds(s * ROWS_PER_TILE, ROWS_PER_TILE)],
            out_hbm.at[a, c, pl.ds(s * ROWS_PER_TILE, ROWS_PER_TILE)],
        )
        if a + 1 < 4:
            plsc.subcore_barrier()


def _make_agg(H):
    @functools.partial(
        pl.kernel,
        out_type=jax.ShapeDtypeStruct((2, H, NP, 128), jnp.float32),
        mesh=_MESH,
        scratch_types=[
            pltpu.VMEM((NCH, CH), jnp.int32),
            pltpu.VMEM((NCH, CH), jnp.int32),
            pltpu.VMEM((CH, 128), jnp.float32),
            pltpu.VMEM_SHARED((NP, 128), jnp.float32),
            pltpu.SemaphoreType.DMA,
        ],
    )
    def agg(src_hbm, dst_hbm, *rest):
        tables = rest[:H]
        out_hbm = rest[H]
        src_v, dst_v, rows, acc, sem = rest[H + 1:]
        c = lax.axis_index("c")
        s = lax.axis_index("s")
        w = c * 16 + s
        pltpu.sync_copy(src_hbm.at[w], src_v)
        pltpu.sync_copy(dst_hbm.at[w], dst_v)
        zero16 = jnp.zeros((16,), dtype=jnp.float32)
        for h in range(H):
            def zbody(i, _):
                for l in range(8):
                    rows[i, pl.ds(l * 16, 16)] = zero16
                return 0

            lax.fori_loop(0, CH, zbody, 0)
            for z in range(ROWS_PER_TILE // CH):
                pltpu.sync_copy(rows, acc.at[pl.ds(s * ROWS_PER_TILE + z * CH, CH)])
            plsc.subcore_barrier()

            table = tables[h]

            def body(j, _):
                pltpu.async_copy(table.at[src_v.at[j]], rows, sem).wait()
                pltpu.sync_copy(rows, acc.at[dst_v.at[j]], add=True)
                return 0

            lax.fori_loop(0, NCH, body, 0)
            plsc.subcore_barrier()
            pltpu.sync_copy(
                acc.at[pl.ds(s * ROWS_PER_TILE, ROWS_PER_TILE)],
                out_hbm.at[c, h, pl.ds(s * ROWS_PER_TILE, ROWS_PER_TILE)],
            )
            if h + 1 < H:
                plsc.subcore_barrier()

    return agg


_agg1 = _make_agg(1)
_agg2 = _make_agg(2)


def _norms_tc(hists):
    RB = 1024

    def body(h_ref, o_ref):
        x = h_ref[...]
        cols = []
        for a in range(4):
            v = jnp.sum(x[a], axis=(0, 2))
            v = jnp.where(v > 0.5, lax.rsqrt(v), 0.0)
            cols.append(v[:, None])
        o_ref[...] = jnp.concatenate(cols, axis=1)

    return pl.pallas_call(
        body,
        grid=(NP // RB,),
        in_specs=[pl.BlockSpec((4, 2, RB, 128), lambda i: (0, 0, i, 0))],
        out_specs=pl.BlockSpec((RB, 4), lambda i: (i, 0)),
        out_shape=jax.ShapeDtypeStruct((NP, 4), jnp.float32),
    )(hists)


def _scale_tc(x, ns):
    RB = 1024

    def body(x_ref, n_ref, o_ref):
        o_ref[...] = x_ref[...] * n_ref[...]

    return pl.pallas_call(
        body,
        grid=(NP // RB,),
        in_specs=[
            pl.BlockSpec((RB, 128), lambda i: (i, 0)),
            pl.BlockSpec((RB, 1), lambda i: (i, 0)),
        ],
        out_specs=pl.BlockSpec((RB, 128), lambda i: (i, 0)),
        out_shape=jax.ShapeDtypeStruct((NP, 128), jnp.float32),
    )(x, ns)


def _layer_tc(P, nd, ns, W):
    H = P.shape[1]
    RB = 1024

    def body(p_ref, nd_ref, ns_ref, w_ref, o0_ref, o1_ref):
        parts = [p_ref[0, h] + p_ref[1, h] for h in range(H)]
        X = parts[0] if H == 1 else jnp.concatenate(parts, axis=1)
        X = X * nd_ref[...]
        Y = jnp.dot(X, w_ref[...], preferred_element_type=jnp.float32)
        Y = jnp.maximum(Y, 0.0) * ns_ref[...]
        o0_ref[...] = Y[:, :128]
        o1_ref[...] = Y[:, 128:]

    return pl.pallas_call(
        body,
        grid=(NP // RB,),
        in_specs=[
            pl.BlockSpec((2, H, RB, 128), lambda i: (0, 0, i, 0)),
            pl.BlockSpec((RB, 1), lambda i: (i, 0)),
            pl.BlockSpec((RB, 1), lambda i: (i, 0)),
            pl.BlockSpec((128 * H, 256), lambda i: (0, 0)),
        ],
        out_specs=[
            pl.BlockSpec((RB, 128), lambda i: (i, 0)),
            pl.BlockSpec((RB, 128), lambda i: (i, 0)),
        ],
        out_shape=[
            jax.ShapeDtypeStruct((NP, 128), jnp.float32),
            jax.ShapeDtypeStruct((NP, 128), jnp.float32),
        ],
    )(P, nd, ns, W)


def _layer2_head_tc(P, nd, ns, W1, Wm, Ws):
    RB = 1024

    def body(p_ref, nd_ref, ns_ref, w1_ref, wm_ref, ws_ref, o_ref):
        X = jnp.concatenate([p_ref[0, 0] + p_ref[1, 0], p_ref[0, 1] + p_ref[1, 1]], axis=1)
        X = X * nd_ref[...]
        h2 = jnp.dot(X, w1_ref[...], preferred_element_type=jnp.float32)
        h2 = jnp.maximum(h2, 0.0) * ns_ref[...]
        tm = jnp.dot(h2, wm_ref[...], preferred_element_type=jnp.float32)
        ts = jnp.dot(h2, ws_ref[...], preferred_element_type=jnp.float32)
        o_ref[...] = jnp.concatenate([tm, ts], axis=1)

    return pl.pallas_call(
        body,
        grid=(NP // RB,),
        in_specs=[
            pl.BlockSpec((2, 2, RB, 128), lambda i: (0, 0, i, 0)),
            pl.BlockSpec((RB, 1), lambda i: (i, 0)),
            pl.BlockSpec((RB, 1), lambda i: (i, 0)),
            pl.BlockSpec((256, 256), lambda i: (0, 0)),
            pl.BlockSpec((256, 64), lambda i: (0, 0)),
            pl.BlockSpec((256, 64), lambda i: (0, 0)),
        ],
        out_specs=pl.BlockSpec((RB, 128), lambda i: (i, 0)),
        out_shape=jax.ShapeDtypeStruct((NP, 128), jnp.float32),
    )(P, nd, ns, W1, Wm, Ws)


def _head_fin_tc(P, nd):
    RB = 1024

    def body(p_ref, nd_ref, om_ref, os_ref):
        X = (p_ref[0, 0] + p_ref[1, 0]) * nd_ref[...]
        om_ref[...] = X[:, :64]
        os_ref[...] = X[:, 64:]

    return pl.pallas_call(
        body,
        grid=(NP // RB,),
        in_specs=[
            pl.BlockSpec((2, 1, RB, 128), lambda i: (0, 0, i, 0)),
            pl.BlockSpec((RB, 1), lambda i: (i, 0)),
        ],
        out_specs=[
            pl.BlockSpec((RB, 64), lambda i: (i, 0)),
            pl.BlockSpec((RB, 64), lambda i: (i, 0)),
        ],
        out_shape=[
            jax.ShapeDtypeStruct((NP, 64), jnp.float32),
            jax.ShapeDtypeStruct((NP, 64), jnp.float32),
        ],
    )(P, nd)


def _final_tc(A_mean, A_std, noise_A, S_mean, S_std, noise_S, Wza, Wzb):
    def body(am, ast, na, sm, sst, nsn, wa, wb, z1_ref, z2_ref):
        def one(mean_ref, std_ref, noise_ref, w_ref, out_ref):
            z = mean_ref[...] + noise_ref[...] * jnp.exp(std_ref[...])
            m = jnp.mean(z, axis=0, keepdims=True)
            d = z - m
            var = jnp.sum(d * d, axis=0, keepdims=True) / (N - 1)
            zn = d * lax.rsqrt(var)
            out_ref[...] = lax.dot_general(
                zn, w_ref[...], (((1,), (1,)), ((), ())),
                preferred_element_type=jnp.float32,
            )

        one(am, ast, na, wa, z1_ref)
        one(sm, sst, nsn, wb, z2_ref)

    return pl.pallas_call(
        body,
        out_shape=[
            jax.ShapeDtypeStruct((N, 64), jnp.float32),
            jax.ShapeDtypeStruct((N, 64), jnp.float32),
        ],
    )(A_mean, A_std, noise_A, S_mean, S_std, noise_S, Wza, Wzb)


def _prep_edges(ei):
    src = ei[0].astype(jnp.int32).reshape(NW, EPW)
    dst = ei[1].astype(jnp.int32).reshape(NW, EPW)
    pad = EPWP - EPW
    src = jnp.pad(src, ((0, 0), (0, pad)), constant_values=GPAD).reshape(NW, NCH, CH)
    dst = jnp.pad(dst, ((0, 0), (0, pad)), constant_values=SPAD).reshape(NW, NCH, CH)
    return src, dst


def kernel(g, adj, features, add_features, Wk0, Wk1, Wk2, Wh0, Wh1, Wh2, Wstd, Wza, Wzb):
    asrc, adst = _prep_edges(adj)
    gsrc, gdst = _prep_edges(g)

    pad = EPWP - EPW

    def dpad(v):
        return jnp.pad(
            v.astype(jnp.int32).reshape(NW, EPW),
            ((0, 0), (0, pad)),
            constant_values=SPAD,
        ).reshape(NW, NCH, CH)

    didx = jnp.stack([dpad(adj[0]), dpad(adj[1]), dpad(g[0]), dpad(g[1])])
    hists = _deg_kernel(didx)
    norms_t = _norms_tc(hists)
    a_ns, a_nd = norms_t[:, 0:1], norms_t[:, 1:2]
    g_ns, g_nd = norms_t[:, 2:3], norms_t[:, 3:4]

    featp = jnp.pad(features, ((0, NP - N), (0, 0)))
    addfp = jnp.pad(add_features, ((0, NP - N), (0, 0)))

    def run_stack(src, dst, ns, nd, x0, W0, W1, W2, Wstd_):
        x0t = _scale_tc(x0, ns)
        P0 = _agg1(src, dst, x0t)
        h1a, h1b = _layer_tc(P0, nd, ns, W0)
        P1 = _agg2(src, dst, h1a, h1b)
        T = _layer2_head_tc(P1, nd, ns, W1, W2, Wstd_)
        P2 = _agg1(src, dst, T)
        mean, std = _head_fin_tc(P2, nd)
        return mean[:N], std[:N]

    A_mean, A_std = run_stack(asrc, adst, a_ns, a_nd, featp, Wk0, Wk1, Wk2, Wstd)
    S_mean, S_std = run_stack(gsrc, gdst, g_ns, g_nd, addfp, Wh0, Wh1, Wh2, Wstd)

    nk = jax.random.key(42)
    noise_A = jax.random.normal(jax.random.fold_in(nk, 0), (N, 64), dtype=jnp.float32)
    noise_S = jax.random.normal(jax.random.fold_in(nk, 1), (N, 64), dtype=jnp.float32)

    z1, z2 = _final_tc(A_mean, A_std, noise_A, S_mean, S_std, noise_S, Wza, Wzb)
    return (z1, z2, A_mean, S_mean, A_std, S_std)


# speedup vs baseline: 1.6002x; 1.4257x over previous
import functools

import jax
import jax.numpy as jnp
from jax import lax
from jax.experimental import pallas as pl
from jax.experimental.pallas import tpu as pltpu
from jax.experimental.pallas import tpu_sc as plsc

N = 10000
E = 320000
NP = 10240
GPAD = NP - 1
SPAD = 10016
NSPAD = 208
NW = 32
EPW = E // NW
CH = 128
NCH = -(-EPW // CH)
EPWP = NCH * CH
ROWS_PER_TILE = NP // 16

_MESH = plsc.VectorSubcoreMesh(core_axis_name="c", subcore_axis_name="s")


@functools.partial(
    pl.kernel,
    out_type=jax.ShapeDtypeStruct((4, 2, NP, 128), jnp.float32),
    mesh=_MESH,
    scratch_types=[
        pltpu.VMEM((NCH, CH), jnp.int32),
        pltpu.VMEM((CH, 128), jnp.float32),
        pltpu.VMEM((CH, 128), jnp.float32),
        pltpu.VMEM_SHARED((NP, 128), jnp.float32),
        pltpu.SemaphoreType.DMA,
    ],
)
def _deg_kernel(didx_hbm, out_hbm, idx_v, ones_rows, zero_rows, acc, dsem):
    c = lax.axis_index("c")
    s = lax.axis_index("s")
    w = c * 16 + s
    ones16 = jnp.full((16,), 1.0 / 128.0, dtype=jnp.float32)
    zero16 = jnp.zeros((16,), dtype=jnp.float32)

    def fill(i, _):
        for l in range(8):
            ones_rows[i, pl.ds(l * 16, 16)] = ones16
            zero_rows[i, pl.ds(l * 16, 16)] = zero16
        return 0

    lax.fori_loop(0, CH, fill, 0)
    K = 8
    for a in range(4):
        pltpu.sync_copy(didx_hbm.at[a, w], idx_v)
        for z in range(ROWS_PER_TILE // CH):
            pltpu.sync_copy(zero_rows, acc.at[pl.ds(s * ROWS_PER_TILE + z * CH, CH)])
        plsc.subcore_barrier()

        def body(gi, _):
            base = gi * K
            for b in range(K):
                pltpu.make_async_copy(
                    ones_rows, acc.at[idx_v.at[base + b]], dsem
                ).start(add=True)
            for b in range(K):
                pltpu.make_async_copy(
                    ones_rows, acc.at[idx_v.at[base + b]], dsem
                ).wait()
            return 0

        lax.fori_loop(0, NCH // K, body, 0)
        for t in range(NCH - (NCH // K) * K):
            j = (NCH // K) * K + t
            pltpu.make_async_copy(ones_rows, acc.at[idx_v.at[j]], dsem).start(add=True)
        for t in range(NCH - (NCH // K) * K):
            j = (NCH // K) * K + t
            pltpu.make_async_copy(ones_rows, acc.at[idx_v.at[j]], dsem).wait()
        plsc.subcore_barrier()
        pltpu.sync_copy(
            acc.at[pl.ds(s * ROWS_PER_TILE, ROWS_PER_TILE)],
            out_hbm.at[a, c, pl.ds(s * ROWS_PER_TILE, ROWS_PER_TILE)],
        )
        if a + 1 < 4:
            plsc.subcore_barrier()


def _make_agg(H):
    @functools.partial(
        pl.kernel,
        out_type=jax.ShapeDtypeStruct((2, H, NP, 128), jnp.float32),
        mesh=_MESH,
        scratch_types=[
            pltpu.VMEM((NCH, CH), jnp.int32),
            pltpu.VMEM((NCH, CH), jnp.int32),
            pltpu.VMEM((CH, 128), jnp.float32),
            pltpu.VMEM_SHARED((NP, 128), jnp.float32),
            pltpu.SemaphoreType.DMA,
        ],
    )
    def agg(src_hbm, dst_hbm, *rest):
        tables = rest[:H]
        out_hbm = rest[H]
        src_v, dst_v, rows, acc, sem = rest[H + 1:]
        c = lax.axis_index("c")
        s = lax.axis_index("s")
        w = c * 16 + s
        pltpu.sync_copy(src_hbm.at[w], src_v)
        pltpu.sync_copy(dst_hbm.at[w], dst_v)
        zero16 = jnp.zeros((16,), dtype=jnp.float32)
        for h in range(H):
            def zbody(i, _):
                for l in range(8):
                    rows[i, pl.ds(l * 16, 16)] = zero16
                return 0

            lax.fori_loop(0, CH, zbody, 0)
            for z in range(ROWS_PER_TILE // CH):
                pltpu.sync_copy(rows, acc.at[pl.ds(s * ROWS_PER_TILE + z * CH, CH)])
            plsc.subcore_barrier()

            table = tables[h]

            def body(j, _):
                pltpu.async_copy(table.at[src_v.at[j]], rows, sem).wait()
                pltpu.sync_copy(rows, acc.at[dst_v.at[j]], add=True)
                return 0

            lax.fori_loop(0, NCH, body, 0)
            plsc.subcore_barrier()
            pltpu.sync_copy(
                acc.at[pl.ds(s * ROWS_PER_TILE, ROWS_PER_TILE)],
                out_hbm.at[c, h, pl.ds(s * ROWS_PER_TILE, ROWS_PER_TILE)],
            )
            if h + 1 < H:
                plsc.subcore_barrier()

    return agg


_agg1 = _make_agg(1)
_agg2 = _make_agg(2)


def _norms_tc(hists):
    RB = 1024

    def body(h_ref, o_ref):
        x = h_ref[...]
        cols = []
        for a in range(4):
            v = jnp.sum(x[a], axis=(0, 2))
            v = jnp.where(v > 0.5, lax.rsqrt(v), 0.0)
            cols.append(v[:, None])
        o_ref[...] = jnp.concatenate(cols, axis=1)

    return pl.pallas_call(
        body,
        grid=(NP // RB,),
        in_specs=[pl.BlockSpec((4, 2, RB, 128), lambda i: (0, 0, i, 0))],
        out_specs=pl.BlockSpec((RB, 4), lambda i: (i, 0)),
        out_shape=jax.ShapeDtypeStruct((NP, 4), jnp.float32),
    )(hists)


def _scale_tc(x, ns):
    RB = 1024

    def body(x_ref, n_ref, o_ref):
        o_ref[...] = x_ref[...] * n_ref[...]

    return pl.pallas_call(
        body,
        grid=(NP // RB,),
        in_specs=[
            pl.BlockSpec((RB, 128), lambda i: (i, 0)),
            pl.BlockSpec((RB, 1), lambda i: (i, 0)),
        ],
        out_specs=pl.BlockSpec((RB, 128), lambda i: (i, 0)),
        out_shape=jax.ShapeDtypeStruct((NP, 128), jnp.float32),
    )(x, ns)


def _layer_tc(P, nd, ns, W):
    H = P.shape[1]
    RB = 1024

    def body(p_ref, nd_ref, ns_ref, w_ref, o0_ref, o1_ref):
        parts = [p_ref[0, h] + p_ref[1, h] for h in range(H)]
        X = parts[0] if H == 1 else jnp.concatenate(parts, axis=1)
        X = X * nd_ref[...]
        Y = jnp.dot(X, w_ref[...], preferred_element_type=jnp.float32)
        Y = jnp.maximum(Y, 0.0) * ns_ref[...]
        o0_ref[...] = Y[:, :128]
        o1_ref[...] = Y[:, 128:]

    return pl.pallas_call(
        body,
        grid=(NP // RB,),
        in_specs=[
            pl.BlockSpec((2, H, RB, 128), lambda i: (0, 0, i, 0)),
            pl.BlockSpec((RB, 1), lambda i: (i, 0)),
            pl.BlockSpec((RB, 1), lambda i: (i, 0)),
            pl.BlockSpec((128 * H, 256), lambda i: (0, 0)),
        ],
        out_specs=[
            pl.BlockSpec((RB, 128), lambda i: (i, 0)),
            pl.BlockSpec((RB, 128), lambda i: (i, 0)),
        ],
        out_shape=[
            jax.ShapeDtypeStruct((NP, 128), jnp.float32),
            jax.ShapeDtypeStruct((NP, 128), jnp.float32),
        ],
    )(P, nd, ns, W)


def _layer2_head_tc(P, nd, ns, W1, Wm, Ws):
    RB = 1024

    def body(p_ref, nd_ref, ns_ref, w1_ref, wm_ref, ws_ref, o_ref):
        X = jnp.concatenate([p_ref[0, 0] + p_ref[1, 0], p_ref[0, 1] + p_ref[1, 1]], axis=1)
        X = X * nd_ref[...]
        h2 = jnp.dot(X, w1_ref[...], preferred_element_type=jnp.float32)
        h2 = jnp.maximum(h2, 0.0) * ns_ref[...]
        tm = jnp.dot(h2, wm_ref[...], preferred_element_type=jnp.float32)
        ts = jnp.dot(h2, ws_ref[...], preferred_element_type=jnp.float32)
        o_ref[...] = jnp.concatenate([tm, ts], axis=1)

    return pl.pallas_call(
        body,
        grid=(NP // RB,),
        in_specs=[
            pl.BlockSpec((2, 2, RB, 128), lambda i: (0, 0, i, 0)),
            pl.BlockSpec((RB, 1), lambda i: (i, 0)),
            pl.BlockSpec((RB, 1), lambda i: (i, 0)),
            pl.BlockSpec((256, 256), lambda i: (0, 0)),
            pl.BlockSpec((256, 64), lambda i: (0, 0)),
            pl.BlockSpec((256, 64), lambda i: (0, 0)),
        ],
        out_specs=pl.BlockSpec((RB, 128), lambda i: (i, 0)),
        out_shape=jax.ShapeDtypeStruct((NP, 128), jnp.float32),
    )(P, nd, ns, W1, Wm, Ws)


def _head_fin_tc(P, nd):
    RB = 1024

    def body(p_ref, nd_ref, om_ref, os_ref):
        X = (p_ref[0, 0] + p_ref[1, 0]) * nd_ref[...]
        om_ref[...] = X[:, :64]
        os_ref[...] = X[:, 64:]

    return pl.pallas_call(
        body,
        grid=(NP // RB,),
        in_specs=[
            pl.BlockSpec((2, 1, RB, 128), lambda i: (0, 0, i, 0)),
            pl.BlockSpec((RB, 1), lambda i: (i, 0)),
        ],
        out_specs=[
            pl.BlockSpec((RB, 64), lambda i: (i, 0)),
            pl.BlockSpec((RB, 64), lambda i: (i, 0)),
        ],
        out_shape=[
            jax.ShapeDtypeStruct((NP, 64), jnp.float32),
            jax.ShapeDtypeStruct((NP, 64), jnp.float32),
        ],
    )(P, nd)


def _final_tc(A_mean, A_std, noise_A, S_mean, S_std, noise_S, Wza, Wzb):
    def body(am, ast, na, sm, sst, nsn, wa, wb, z1_ref, z2_ref):
        def one(mean_ref, std_ref, noise_ref, w_ref, out_ref):
            z = mean_ref[...] + noise_ref[...] * jnp.exp(std_ref[...])
            m = jnp.mean(z, axis=0, keepdims=True)
            d = z - m
            var = jnp.sum(d * d, axis=0, keepdims=True) / (N - 1)
            zn = d * lax.rsqrt(var)
            out_ref[...] = lax.dot_general(
                zn, w_ref[...], (((1,), (1,)), ((), ())),
                preferred_element_type=jnp.float32,
            )

        one(am, ast, na, wa, z1_ref)
        one(sm, sst, nsn, wb, z2_ref)

    return pl.pallas_call(
        body,
        out_shape=[
            jax.ShapeDtypeStruct((N, 64), jnp.float32),
            jax.ShapeDtypeStruct((N, 64), jnp.float32),
        ],
    )(A_mean, A_std, noise_A, S_mean, S_std, noise_S, Wza, Wzb)


def _spad_block(pad):
    v = SPAD + jnp.arange(pad, dtype=jnp.int32) % NSPAD
    return jnp.broadcast_to(v, (NW, pad))


def _prep_edges(ei):
    src = ei[0].astype(jnp.int32).reshape(NW, EPW)
    dst = ei[1].astype(jnp.int32).reshape(NW, EPW)
    pad = EPWP - EPW
    src = jnp.pad(src, ((0, 0), (0, pad)), constant_values=GPAD).reshape(NW, NCH, CH)
    dst = jnp.concatenate([dst, _spad_block(pad)], axis=1).reshape(NW, NCH, CH)
    return src, dst


def kernel(g, adj, features, add_features, Wk0, Wk1, Wk2, Wh0, Wh1, Wh2, Wstd, Wza, Wzb):
    asrc, adst = _prep_edges(adj)
    gsrc, gdst = _prep_edges(g)

    pad = EPWP - EPW

    def dpad(v):
        return jnp.concatenate(
            [v.astype(jnp.int32).reshape(NW, EPW), _spad_block(pad)], axis=1
        ).reshape(NW, NCH, CH)

    didx = jnp.stack([dpad(adj[0]), dpad(adj[1]), dpad(g[0]), dpad(g[1])])
    hists = _deg_kernel(didx)
    norms_t = _norms_tc(hists)
    a_ns, a_nd = norms_t[:, 0:1], norms_t[:, 1:2]
    g_ns, g_nd = norms_t[:, 2:3], norms_t[:, 3:4]

    featp = jnp.pad(features, ((0, NP - N), (0, 0)))
    addfp = jnp.pad(add_features, ((0, NP - N), (0, 0)))

    def run_stack(src, dst, ns, nd, x0, W0, W1, W2, Wstd_):
        x0t = _scale_tc(x0, ns)
        P0 = _agg1(src, dst, x0t)
        h1a, h1b = _layer_tc(P0, nd, ns, W0)
        P1 = _agg2(src, dst, h1a, h1b)
        T = _layer2_head_tc(P1, nd, ns, W1, W2, Wstd_)
        P2 = _agg1(src, dst, T)
        mean, std = _head_fin_tc(P2, nd)
        return mean[:N], std[:N]

    A_mean, A_std = run_stack(asrc, adst, a_ns, a_nd, featp, Wk0, Wk1, Wk2, Wstd)
    S_mean, S_std = run_stack(gsrc, gdst, g_ns, g_nd, addfp, Wh0, Wh1, Wh2, Wstd)

    nk = jax.random.key(42)
    noise_A = jax.random.normal(jax.random.fold_in(nk, 0), (N, 64), dtype=jnp.float32)
    noise_S = jax.random.normal(jax.random.fold_in(nk, 1), (N, 64), dtype=jnp.float32)

    z1, z2 = _final_tc(A_mean, A_std, noise_A, S_mean, S_std, noise_S, Wza, Wzb)
    return (z1, z2, A_mean, S_mean, A_std, S_std)
